# baseline (speedup 1.0000x reference)
import jax
import jax.numpy as jnp
from jax import lax
from jax.experimental import pallas as pl
from jax.experimental.pallas import tpu as pltpu
from jax.experimental.pallas import tpu_sc as plsc

B = 1024
M = 50
S = 20
V = 100000
E = 64
EA = 4 * E

CH = 16
G = 4
GS = CH * S // G

NCHK = 2
BH = B // NCHK


def _sc_pool_body(story_idx, q_idx, tab,
                  msum, qsum,
                  idx_v, rows_v, out_v,
                  gsem0, gsem1, ssem0, ssem1):
    wid = lax.axis_index("s") * 2 + lax.axis_index("c")
    gsems = (gsem0, gsem1)
    ssems = (ssem0, ssem1)

    def fire(idx_hbm, st, slot):
        pltpu.sync_copy(idx_hbm.at[wid, st], idx_v.at[slot])
        for g in range(G):
            pltpu.async_copy(tab.at[idx_v.at[slot, g]],
                             rows_v.at[slot, pl.ds(g * GS, GS)],
                             gsems[slot])

    def drain_gather(slot):
        for g in range(G):
            pltpu.make_async_copy(tab.at[pl.ds(0, GS)],
                                  rows_v.at[slot, pl.ds(g * GS, GS)],
                                  gsems[slot]).wait()

    def accum_store(out_hbm, seg_base, slot, drain_prev):
        dst = out_hbm.at[pl.ds(seg_base, CH)]
        @pl.when(drain_prev)
        def _():
            pltpu.make_async_copy(out_v.at[slot], dst, ssems[slot]).wait()

        def seg_body(c, carry):
            base = c * S
            for cg in range(EA // 32):
                sl = pl.ds(cg * 16, 16)

                def halves(r):
                    w = rows_v[slot, r, sl]
                    ev = plsc.bitcast(lax.shift_left(w, 16), jnp.float32)
                    od = plsc.bitcast(lax.bitwise_and(w, -65536),
                                      jnp.float32)
                    return ev, od

                acc_e, acc_o = halves(base)
                for s2 in range(1, S):
                    ev, od = halves(base + s2)
                    acc_e = acc_e + ev
                    acc_o = acc_o + od
                base_col = 64 * (cg // 2) + (cg % 2) * 16
                out_v[slot, c, pl.ds(base_col, 16)] = acc_e
                out_v[slot, c, pl.ds(base_col + 32, 16)] = acc_o
            return carry

        lax.fori_loop(0, CH, seg_body, 0)
        pltpu.async_copy(out_v.at[slot], dst, ssems[slot])

    n_steps = story_idx.shape[1]
    nq_steps = q_idx.shape[1]

    fire(story_idx, 0, 0)

    def outer_body(st0, carry):
        @pl.when(st0 + 1 < n_steps)
        def _():
            fire(story_idx, st0 + 1, 1)

        drain_gather(0)
        accum_store(msum, (wid * n_steps + st0) * CH, 0, st0 >= 2)

        @pl.when(st0 + 2 < n_steps)
        def _():
            fire(story_idx, st0 + 2, 0)

        drain_gather(1)
        accum_store(msum, (wid * n_steps + st0 + 1) * CH, 1, st0 >= 2)
        return carry

    lax.fori_loop(0, n_steps // 2, lambda i, c: outer_body(i * 2, c), 0)

    def qstep_body(st, carry):
        fire(q_idx, st, 0)
        drain_gather(0)
        accum_store(qsum, (wid * nq_steps + st) * CH, 0, st >= 0)
        return carry

    lax.fori_loop(0, nq_steps, qstep_body, 0)

    pltpu.make_async_copy(out_v.at[0], qsum.at[pl.ds(0, CH)], ssems[0]).wait()
    pltpu.make_async_copy(out_v.at[1], msum.at[pl.ds(0, CH)], ssems[1]).wait()


def _hops_body(q_ref, mall_ref, u_ref):
    mall = mall_ref[...]
    u = q_ref[...][:, :E]
    ms = tuple(mall[:, :, i * E:(i + 1) * E] for i in range(4))
    for i in range(3):
        m, c = ms[i], ms[i + 1]
        lg = jnp.sum(m * u[:, None, :], axis=2)
        lg = lg - jnp.max(lg, axis=1, keepdims=True)
        ex = jnp.exp(lg)
        p = ex / jnp.sum(ex, axis=1, keepdims=True)
        u = jnp.sum(c * p[:, :, None], axis=1) + u
    u_ref[...] = u


BT = 256
VT = 5000
NB = B // BT
NBH = BH // BT
NV = V // VT


def _sumexp_body(u_ref, e_ref, s_ref):
    v = pl.program_id(0)
    b = pl.program_id(1)
    lg = lax.dot_general(u_ref[...], e_ref[...], (((1,), (1,)), ((), ())),
                         preferred_element_type=jnp.float32)
    part = jnp.sum(jnp.exp(lg), axis=1)[None, :]

    @pl.when(v == 0)
    def _():
        s_ref[pl.ds(b, 1), :] = part

    @pl.when(v != 0)
    def _():
        s_ref[pl.ds(b, 1), :] = s_ref[pl.ds(b, 1), :] + part


VT2 = 6272
NV2 = 15
VTAIL = V - NV2 * VT2


def _normexp_body(u_ref, e_ref, s_ref, o_hbm, buf_ref, tbuf_ref, sem, tsem):
    v = pl.program_id(0)
    b = pl.program_id(1)
    step = v * NB + b
    slot = step % 2
    nmain = NV2 * NB
    lg = lax.dot_general(u_ref[...], e_ref[...], (((1,), (1,)), ((), ())),
                         preferred_element_type=jnp.float32)
    sc = s_ref[pl.ds(b, 1), :]
    res = jnp.exp(lg) * (1.0 / sc[0, :])[:, None]

    @pl.when((step >= 2) & (step < nmain + 2))
    def _():
        d = o_hbm.at[pl.ds(b * BT, BT), pl.ds(0, VT2)]
        pltpu.make_async_copy(buf_ref.at[slot], d, sem).wait()

    @pl.when(step >= nmain + 2)
    def _():
        d = o_hbm.at[pl.ds(b * BT, BT), pl.ds(NV2 * VT2, VTAIL)]
        pltpu.make_async_copy(tbuf_ref.at[slot], d, tsem).wait()

    @pl.when(v < NV2)
    def _():
        buf_ref[pl.ds(slot, 1), :, :] = res[None]
        dst = o_hbm.at[pl.ds(b * BT, BT), pl.ds(v * VT2, VT2)]
        pltpu.make_async_copy(buf_ref.at[slot], dst, sem).start()

    @pl.when(v == NV2)
    def _():
        lg_t = lax.dot_general(u_ref[...], e_ref[pl.ds(0, VTAIL), :],
                               (((1,), (1,)), ((), ())),
                               preferred_element_type=jnp.float32)
        tbuf_ref[pl.ds(slot, 1), :, :] = (
            jnp.exp(lg_t) * (1.0 / sc[0, :])[:, None])[None]
        dst = o_hbm.at[pl.ds(b * BT, BT), pl.ds(NV2 * VT2, VTAIL)]
        pltpu.make_async_copy(tbuf_ref.at[slot], dst, tsem).start()

    @pl.when(step == (NV2 + 1) * NB - 1)
    def _():
        for sl2 in range(2):
            d = o_hbm.at[pl.ds(b * BT, BT), pl.ds(NV2 * VT2, VTAIL)]
            pltpu.make_async_copy(tbuf_ref.at[sl2], d, tsem).wait()


def kernel(story, question, emb_0, emb_1, emb_2, emb_3):
    story = story.astype(jnp.int32)
    question = question.astype(jnp.int32)
    tab = jnp.concatenate(
        [e.astype(jnp.bfloat16) for e in (emb_0, emb_1, emb_2, emb_3)],
        axis=1)
    tab = lax.bitcast_convert_type(tab.reshape(V, EA // 2, 2), jnp.int32)

    emb_3p = jnp.concatenate([emb_3[:, 0::2], emb_3[:, 1::2]], axis=1)

    info = plsc.get_sparse_core_info()
    nw = info.num_cores * info.num_subcores

    n_steps = (BH * M) // (nw * CH)
    nq_steps = BH // (nw * CH)
    story_idx = story.reshape(NCHK, nw, n_steps, G, GS)
    q_idx = question.reshape(NCHK, nw, nq_steps, G, GS)

    mesh = plsc.VectorSubcoreMesh(core_axis_name="c", subcore_axis_name="s")
    sc_pool = pl.kernel(
        _sc_pool_body,
        out_type=(
            jax.ShapeDtypeStruct((BH * M, EA), jnp.float32),
            jax.ShapeDtypeStruct((BH, EA), jnp.float32),
        ),
        mesh=mesh,
        compiler_params=pltpu.CompilerParams(needs_layout_passes=False),
        scratch_types=[
            pltpu.VMEM((2, G, GS), jnp.int32),
            pltpu.VMEM((2, CH * S, EA // 2), jnp.int32),
            pltpu.VMEM((2, CH, EA), jnp.float32),
            pltpu.SemaphoreType.DMA,
            pltpu.SemaphoreType.DMA,
            pltpu.SemaphoreType.DMA,
            pltpu.SemaphoreType.DMA,
        ],
    )
    pooled = [sc_pool(story_idx[ci], q_idx[ci], tab) for ci in range(NCHK)]

    bt_h = 128
    us = []
    ss = []
    for ci in range(NCHK):
        msum, qsum = pooled[ci]
        mall = msum.reshape(BH, M, EA)
        u = pl.pallas_call(
            _hops_body,
            grid=(BH // bt_h,),
            in_specs=[
                pl.BlockSpec((bt_h, EA), lambda i: (i, 0)),
                pl.BlockSpec((bt_h, M, EA), lambda i: (i, 0, 0)),
            ],
            out_specs=pl.BlockSpec((bt_h, E), lambda i: (i, 0)),
            out_shape=jax.ShapeDtypeStruct((BH, E), jnp.float32),
        )(qsum, mall)

        s = pl.pallas_call(
            _sumexp_body,
            grid=(NV, NBH),
            in_specs=[
                pl.BlockSpec((BT, E), lambda v, b: (b, 0)),
                pl.BlockSpec((VT, E), lambda v, b: (v, 0)),
            ],
            out_specs=pl.BlockSpec((NBH, BT), lambda v, b: (0, 0)),
            out_shape=jax.ShapeDtypeStruct((NBH, BT), jnp.float32),
        )(u, emb_3p)
        us.append(u)
        ss.append(s)

    u_all = jnp.concatenate(us, axis=0)
    s_all = jnp.concatenate(ss, axis=0)

    out = pl.pallas_call(
        _normexp_body,
        grid=(NV2 + 1, NB),
        in_specs=[
            pl.BlockSpec((BT, E), lambda v, b: (b, 0)),
            pl.BlockSpec((VT2, E), lambda v, b: (v, 0)),
            pl.BlockSpec((NB, BT), lambda v, b: (0, 0)),
        ],
        out_specs=pl.BlockSpec(memory_space=pl.ANY),
        out_shape=jax.ShapeDtypeStruct((B, V), jnp.float32),
        scratch_shapes=[
            pltpu.VMEM((2, BT, VT2), jnp.float32),
            pltpu.VMEM((2, BT, VTAIL), jnp.float32),
            pltpu.SemaphoreType.DMA,
            pltpu.SemaphoreType.DMA,
        ],
    )(u_all, emb_3p, s_all)
    return out


# speedup vs baseline: 1.3416x; 1.3416x over previous
import jax
import jax.numpy as jnp
from jax import lax
from jax.experimental import pallas as pl
from jax.experimental.pallas import tpu as pltpu
from jax.experimental.pallas import tpu_sc as plsc

B = 1024
M = 50
S = 20
V = 100000
E = 64
EA = 4 * E

CH = 16
G = 4
GS = CH * S // G

NCHK = 2
BH = B // NCHK


def _sc_pool_body(story_idx, q_idx, tab,
                  msum, qsum,
                  idx_v, rows_v, out_v,
                  gsem0, gsem1, ssem0, ssem1):
    wid = lax.axis_index("s") * 2 + lax.axis_index("c")
    gsems = (gsem0, gsem1)
    ssems = (ssem0, ssem1)

    def fire(idx_hbm, st, slot):
        pltpu.sync_copy(idx_hbm.at[wid, st], idx_v.at[slot])
        for g in range(G):
            pltpu.async_copy(tab.at[idx_v.at[slot, g]],
                             rows_v.at[slot, pl.ds(g * GS, GS)],
                             gsems[slot])

    def drain_gather(slot):
        for g in range(G):
            pltpu.make_async_copy(tab.at[pl.ds(0, GS)],
                                  rows_v.at[slot, pl.ds(g * GS, GS)],
                                  gsems[slot]).wait()

    def accum_store(out_hbm, seg_base, slot, drain_prev):
        dst = out_hbm.at[pl.ds(seg_base, CH)]
        @pl.when(drain_prev)
        def _():
            pltpu.make_async_copy(out_v.at[slot], dst, ssems[slot]).wait()

        def seg_body(c, carry):
            base = c * S
            for cg in range(EA // 32):
                sl = pl.ds(cg * 16, 16)

                def halves(r):
                    w = rows_v[slot, r, sl]
                    ev = plsc.bitcast(lax.shift_left(w, 16), jnp.float32)
                    od = plsc.bitcast(lax.bitwise_and(w, -65536),
                                      jnp.float32)
                    return ev, od

                acc_e, acc_o = halves(base)
                for s2 in range(1, S):
                    ev, od = halves(base + s2)
                    acc_e = acc_e + ev
                    acc_o = acc_o + od
                base_col = 64 * (cg // 2) + (cg % 2) * 16
                out_v[slot, c, pl.ds(base_col, 16)] = acc_e
                out_v[slot, c, pl.ds(base_col + 32, 16)] = acc_o
            return carry

        lax.fori_loop(0, CH, seg_body, 0)
        pltpu.async_copy(out_v.at[slot], dst, ssems[slot])

    n_steps = story_idx.shape[1]
    nq_steps = q_idx.shape[1]

    fire(story_idx, 0, 0)

    def outer_body(st0, carry):
        @pl.when(st0 + 1 < n_steps)
        def _():
            fire(story_idx, st0 + 1, 1)

        drain_gather(0)
        accum_store(msum, (wid * n_steps + st0) * CH, 0, st0 >= 2)

        @pl.when(st0 + 2 < n_steps)
        def _():
            fire(story_idx, st0 + 2, 0)

        drain_gather(1)
        accum_store(msum, (wid * n_steps + st0 + 1) * CH, 1, st0 >= 2)
        return carry

    lax.fori_loop(0, n_steps // 2, lambda i, c: outer_body(i * 2, c), 0)

    def qstep_body(st, carry):
        fire(q_idx, st, 0)
        drain_gather(0)
        accum_store(qsum, (wid * nq_steps + st) * CH, 0, st >= 0)
        return carry

    lax.fori_loop(0, nq_steps, qstep_body, 0)

    pltpu.make_async_copy(out_v.at[0], qsum.at[pl.ds(0, CH)], ssems[0]).wait()
    pltpu.make_async_copy(out_v.at[1], msum.at[pl.ds(0, CH)], ssems[1]).wait()


def _pack_body(e0_ref, e1_ref, e2_ref, e3_ref, t_ref):
    def b16(x):
        u = lax.bitcast_convert_type(x, jnp.int32)
        r = u + 0x7FFF + (lax.shift_right_logical(u, 16) & 1)
        return lax.shift_right_logical(r, 16)

    for t, e_ref in enumerate((e0_ref, e1_ref, e2_ref, e3_ref)):
        e = e_ref[...]
        lo = b16(e[:, :E // 2])
        hi = b16(e[:, E // 2:])
        t_ref[:, 32 * t:32 * (t + 1)] = lax.shift_left(hi, 16) | lo


def _hops_body(q_ref, mall_ref, u_ref):
    mall = mall_ref[...]
    u = q_ref[...][:, :E]
    ms = tuple(mall[:, :, i * E:(i + 1) * E] for i in range(4))
    for i in range(3):
        m, c = ms[i], ms[i + 1]
        lg = jnp.sum(m * u[:, None, :], axis=2)
        lg = lg - jnp.max(lg, axis=1, keepdims=True)
        ex = jnp.exp(lg)
        p = ex / jnp.sum(ex, axis=1, keepdims=True)
        u = jnp.sum(c * p[:, :, None], axis=1) + u
    u_ref[...] = u


BT = 256
VT = 5000
NB = B // BT
NBH = BH // BT
NV = V // VT


def _sumexp_body(u_ref, e_ref, s_ref):
    v = pl.program_id(0)
    b = pl.program_id(1)
    lg = lax.dot_general(u_ref[...], e_ref[...], (((1,), (1,)), ((), ())),
                         preferred_element_type=jnp.float32)
    part = jnp.sum(jnp.exp(lg), axis=1)[None, :]

    @pl.when(v == 0)
    def _():
        s_ref[pl.ds(b, 1), :] = part

    @pl.when(v != 0)
    def _():
        s_ref[pl.ds(b, 1), :] = s_ref[pl.ds(b, 1), :] + part


VT2 = 6272
NV2 = 15
VTAIL = V - NV2 * VT2


def _normexp_body(u_ref, e_ref, s_ref, o_hbm, buf_ref, tbuf_ref, sem, tsem):
    v = pl.program_id(0)
    b = pl.program_id(1)
    step = v * NB + b
    slot = step % 2
    nmain = NV2 * NB
    lg = lax.dot_general(u_ref[...], e_ref[...], (((1,), (1,)), ((), ())),
                         preferred_element_type=jnp.float32)
    sc = s_ref[pl.ds(b, 1), :]
    res = jnp.exp(lg) * (1.0 / sc[0, :])[:, None]

    @pl.when((step >= 2) & (step < nmain + 2))
    def _():
        d = o_hbm.at[pl.ds(b * BT, BT), pl.ds(0, VT2)]
        pltpu.make_async_copy(buf_ref.at[slot], d, sem).wait()

    @pl.when(step >= nmain + 2)
    def _():
        d = o_hbm.at[pl.ds(b * BT, BT), pl.ds(NV2 * VT2, VTAIL)]
        pltpu.make_async_copy(tbuf_ref.at[slot], d, tsem).wait()

    @pl.when(v < NV2)
    def _():
        buf_ref[pl.ds(slot, 1), :, :] = res[None]
        dst = o_hbm.at[pl.ds(b * BT, BT), pl.ds(v * VT2, VT2)]
        pltpu.make_async_copy(buf_ref.at[slot], dst, sem).start()

    @pl.when(v == NV2)
    def _():
        lg_t = lax.dot_general(u_ref[...], e_ref[pl.ds(0, VTAIL), :],
                               (((1,), (1,)), ((), ())),
                               preferred_element_type=jnp.float32)
        tbuf_ref[pl.ds(slot, 1), :, :] = (
            jnp.exp(lg_t) * (1.0 / sc[0, :])[:, None])[None]
        dst = o_hbm.at[pl.ds(b * BT, BT), pl.ds(NV2 * VT2, VTAIL)]
        pltpu.make_async_copy(tbuf_ref.at[slot], dst, tsem).start()

    @pl.when(step == (NV2 + 1) * NB - 1)
    def _():
        for sl2 in range(2):
            d = o_hbm.at[pl.ds(b * BT, BT), pl.ds(NV2 * VT2, VTAIL)]
            pltpu.make_async_copy(tbuf_ref.at[sl2], d, tsem).wait()


def kernel(story, question, emb_0, emb_1, emb_2, emb_3):
    story = story.astype(jnp.int32)
    question = question.astype(jnp.int32)
    vtc = 2000
    tab = pl.pallas_call(
        _pack_body,
        grid=(V // vtc,),
        in_specs=[pl.BlockSpec((vtc, E), lambda i: (i, 0))] * 4,
        out_specs=pl.BlockSpec((vtc, EA // 2), lambda i: (i, 0)),
        out_shape=jax.ShapeDtypeStruct((V, EA // 2), jnp.int32),
    )(emb_0, emb_1, emb_2, emb_3)

    info = plsc.get_sparse_core_info()
    nw = info.num_cores * info.num_subcores

    n_steps = (BH * M) // (nw * CH)
    nq_steps = BH // (nw * CH)
    story_idx = story.reshape(NCHK, nw, n_steps, G, GS)
    q_idx = question.reshape(NCHK, nw, nq_steps, G, GS)

    mesh = plsc.VectorSubcoreMesh(core_axis_name="c", subcore_axis_name="s")
    sc_pool = pl.kernel(
        _sc_pool_body,
        out_type=(
            jax.ShapeDtypeStruct((BH * M, EA), jnp.float32),
            jax.ShapeDtypeStruct((BH, EA), jnp.float32),
        ),
        mesh=mesh,
        compiler_params=pltpu.CompilerParams(needs_layout_passes=False),
        scratch_types=[
            pltpu.VMEM((2, G, GS), jnp.int32),
            pltpu.VMEM((2, CH * S, EA // 2), jnp.int32),
            pltpu.VMEM((2, CH, EA), jnp.float32),
            pltpu.SemaphoreType.DMA,
            pltpu.SemaphoreType.DMA,
            pltpu.SemaphoreType.DMA,
            pltpu.SemaphoreType.DMA,
        ],
    )
    pooled = [sc_pool(story_idx[ci], q_idx[ci], tab) for ci in range(NCHK)]

    bt_h = 128
    us = []
    ss = []
    for ci in range(NCHK):
        msum, qsum = pooled[ci]
        mall = msum.reshape(BH, M, EA)
        u = pl.pallas_call(
            _hops_body,
            grid=(BH // bt_h,),
            in_specs=[
                pl.BlockSpec((bt_h, EA), lambda i: (i, 0)),
                pl.BlockSpec((bt_h, M, EA), lambda i: (i, 0, 0)),
            ],
            out_specs=pl.BlockSpec((bt_h, E), lambda i: (i, 0)),
            out_shape=jax.ShapeDtypeStruct((BH, E), jnp.float32),
        )(qsum, mall)

        s = pl.pallas_call(
            _sumexp_body,
            grid=(NV, NBH),
            in_specs=[
                pl.BlockSpec((BT, E), lambda v, b: (b, 0)),
                pl.BlockSpec((VT, E), lambda v, b: (v, 0)),
            ],
            out_specs=pl.BlockSpec((NBH, BT), lambda v, b: (0, 0)),
            out_shape=jax.ShapeDtypeStruct((NBH, BT), jnp.float32),
        )(u, emb_3)
        us.append(u)
        ss.append(s)

    u_all = jnp.concatenate(us, axis=0)
    s_all = jnp.concatenate(ss, axis=0)

    out = pl.pallas_call(
        _normexp_body,
        grid=(NV2 + 1, NB),
        in_specs=[
            pl.BlockSpec((BT, E), lambda v, b: (b, 0)),
            pl.BlockSpec((VT2, E), lambda v, b: (v, 0)),
            pl.BlockSpec((NB, BT), lambda v, b: (0, 0)),
        ],
        out_specs=pl.BlockSpec(memory_space=pltpu.MemorySpace.HBM),
        out_shape=jax.ShapeDtypeStruct((B, V), jnp.float32),
        scratch_shapes=[
            pltpu.VMEM((2, BT, VT2), jnp.float32),
            pltpu.VMEM((2, BT, VTAIL), jnp.float32),
            pltpu.SemaphoreType.DMA,
            pltpu.SemaphoreType.DMA,
        ],
    )(u_all, emb_3, s_all)
    return out
